# baseline (speedup 1.0000x reference)
import functools

import jax
import jax.numpy as jnp
from jax import lax
from jax.experimental import pallas as pl
from jax.experimental.pallas import tpu as pltpu
from jax.experimental.pallas import tpu_sc as plsc

N_NODES = 10000
N_EDGES = 320000
D_IN = 128
D_OUT = 128
DP = 144

NC = 2
NS = 16
NW = NC * NS
EDGE_B = 80
E_PER_W = 10240
E_TOTAL_PAD = E_PER_W * NW
NB = E_PER_W // EDGE_B
N_PAD = 10240
ROWS_PER_TILE = N_PAD // NS
N_SPARE = N_PAD - N_NODES


@functools.cache
def _build_sc_scatter():
    mesh = plsc.VectorSubcoreMesh(core_axis_name="c", subcore_axis_name="s",
                                  num_cores=NC, num_subcores=NS)

    @functools.partial(
        pl.kernel,
        out_type=jax.ShapeDtypeStruct((NC, N_PAD, DP), jnp.float32),
        mesh=mesh,
        scratch_types=[
            pltpu.VMEM((EDGE_B,), jnp.int32),
            pltpu.VMEM((EDGE_B,), jnp.int32),
            pltpu.VMEM((EDGE_B, DP), jnp.float32),
            pltpu.VMEM((EDGE_B,), jnp.int32),
            pltpu.VMEM((EDGE_B,), jnp.int32),
            pltpu.VMEM((EDGE_B, DP), jnp.float32),
            pltpu.VMEM_SHARED((N_PAD, DP), jnp.float32),
            pltpu.SemaphoreType.DMA,
            pltpu.SemaphoreType.DMA,
        ],
        compiler_params=pltpu.CompilerParams(use_tc_tiling_on_sc=False),
    )
    def _sc_scatter(xa_hbm, col_hbm, row_hbm, zeros_hbm, out_hbm,
                    cola, rowa, bufa, colb, rowb, bufb, agg_sh,
                    sema, semb):
        cid = lax.axis_index("c")
        sid = lax.axis_index("s")
        w = cid * NS + sid
        r0 = sid * ROWS_PER_TILE
        ebase = w * E_PER_W
        pltpu.sync_copy(zeros_hbm, agg_sh.at[pl.ds(r0, ROWS_PER_TILE)])
        plsc.subcore_barrier()

        pltpu.sync_copy(col_hbm.at[pl.ds(ebase, EDGE_B)], cola)
        pltpu.sync_copy(row_hbm.at[pl.ds(ebase, EDGE_B)], rowa)
        pltpu.async_copy(xa_hbm.at[cola], bufa, sema)

        slots = ((cola, rowa, bufa, sema), (colb, rowb, bufb, semb))

        def chunk(c, carry):
            for j in range(2):
                g = 2 * c + j
                cv, rv, bf, sm = slots[j]
                cv2, rv2, bf2, sm2 = slots[1 - j]

                @pl.when(g + 1 < NB)
                def _():
                    nb = ebase + g * EDGE_B + EDGE_B
                    pltpu.sync_copy(col_hbm.at[pl.ds(nb, EDGE_B)], cv2)
                    pltpu.sync_copy(row_hbm.at[pl.ds(nb, EDGE_B)], rv2)
                    pltpu.async_copy(xa_hbm.at[cv2], bf2, sm2)

                pltpu.make_async_copy(xa_hbm.at[cv], bf, sm).wait()
                pltpu.sync_copy(bf, agg_sh.at[rv], add=True)
            return carry

        lax.fori_loop(0, NB // 2, chunk, 0)
        plsc.subcore_barrier()
        pltpu.sync_copy(agg_sh.at[pl.ds(r0, ROWS_PER_TILE)],
                        out_hbm.at[cid, pl.ds(r0, ROWS_PER_TILE)])

    return _sc_scatter


_TC_R = 1000


def _tc_body(x_ref, p0_ref, p1_ref, wt_ref, b_ref, o_ref):
    s = p0_ref[0, :, :D_IN] + p1_ref[0, :, :D_IN]
    deg = p0_ref[0, :, D_IN:D_IN + 1] + p1_ref[0, :, D_IN:D_IN + 1]
    agg = s / jnp.maximum(deg, 1.0)
    out = jnp.dot(x_ref[...], wt_ref[:D_IN, :],
                  preferred_element_type=jnp.float32)
    out += jnp.dot(agg, wt_ref[D_IN:, :], preferred_element_type=jnp.float32)
    o_ref[...] = out + b_ref[...]


def kernel(x, edge_index, W, b):
    ei = edge_index.astype(jnp.int32)
    pad = E_TOTAL_PAD - N_EDGES
    dummy_rows = N_NODES + jnp.arange(pad, dtype=jnp.int32) % N_SPARE
    row = jnp.concatenate([ei[0], dummy_rows])
    col = jnp.concatenate([ei[1], jnp.zeros((pad,), jnp.int32)])
    ones_pad = jnp.concatenate(
        [jnp.ones((N_NODES, 1), jnp.float32),
         jnp.zeros((N_NODES, DP - D_IN - 1), jnp.float32)], axis=1)
    xa = jnp.concatenate([x.astype(jnp.float32), ones_pad], axis=1)
    zeros = jnp.zeros((ROWS_PER_TILE, DP), jnp.float32)

    partials = _build_sc_scatter()(xa, col, row, zeros)

    wt = W.T.astype(jnp.float32)
    b2 = b.reshape(1, D_OUT).astype(jnp.float32)
    grid = (N_NODES // _TC_R,)
    return pl.pallas_call(
        _tc_body,
        grid=grid,
        in_specs=[
            pl.BlockSpec((_TC_R, D_IN), lambda i: (i, 0)),
            pl.BlockSpec((1, _TC_R, DP), lambda i: (0, i, 0)),
            pl.BlockSpec((1, _TC_R, DP), lambda i: (1, i, 0)),
            pl.BlockSpec((2 * D_IN, D_OUT), lambda i: (0, 0)),
            pl.BlockSpec((1, D_OUT), lambda i: (0, 0)),
        ],
        out_specs=pl.BlockSpec((_TC_R, D_OUT), lambda i: (i, 0)),
        out_shape=jax.ShapeDtypeStruct((N_NODES, D_OUT), jnp.float32),
    )(x.astype(jnp.float32), partials, partials, wt, b2)


# speedup vs baseline: 3.1103x; 3.1103x over previous
import functools

import jax
import jax.numpy as jnp
from jax import lax
from jax.experimental import pallas as pl
from jax.experimental.pallas import tpu as pltpu
from jax.experimental.pallas import tpu_sc as plsc

N_NODES = 10000
N_EDGES = 320000
D_IN = 128
D_OUT = 128
DP = 144

NC = 2
NS = 16
NW = NC * NS
EDGE_B = 64
E_PER_W = 10240
E_TOTAL_PAD = E_PER_W * NW
NB = E_PER_W // EDGE_B
NSLOT = 4
GDEPTH = 2
N_PAD = 10240
ROWS_PER_TILE = N_PAD // NS
N_SPARE = N_PAD - N_NODES


@functools.cache
def _build_sc_scatter():
    mesh = plsc.VectorSubcoreMesh(core_axis_name="c", subcore_axis_name="s",
                                  num_cores=NC, num_subcores=NS)

    @functools.partial(
        pl.kernel,
        out_type=jax.ShapeDtypeStruct((NC, N_PAD, DP), jnp.float32),
        mesh=mesh,
        scratch_types=[
            pltpu.VMEM((2, EDGE_B), jnp.int32),
            pltpu.VMEM((2, EDGE_B), jnp.int32),
            pltpu.VMEM((2, EDGE_B), jnp.int32),
            pltpu.VMEM((2, EDGE_B), jnp.int32),
            pltpu.VMEM((EDGE_B, DP), jnp.float32),
            pltpu.VMEM((EDGE_B, DP), jnp.float32),
            pltpu.VMEM((EDGE_B, DP), jnp.float32),
            pltpu.VMEM((EDGE_B, DP), jnp.float32),
            pltpu.VMEM_SHARED((N_PAD, DP), jnp.float32),
            pltpu.SemaphoreType.DMA,
            pltpu.SemaphoreType.DMA,
            pltpu.SemaphoreType.DMA,
            pltpu.SemaphoreType.DMA,
            pltpu.SemaphoreType.DMA,
            pltpu.SemaphoreType.DMA,
            pltpu.SemaphoreType.DMA,
            pltpu.SemaphoreType.DMA,
        ],
        compiler_params=pltpu.CompilerParams(use_tc_tiling_on_sc=False),
    )
    def _sc_scatter(xa_hbm, idx_hbm, zeros_hbm, out_hbm,
                    i0, i1, i2, i3, b0, b1, b2, b3, agg_sh,
                    g0, g1, g2, g3, s0, s1, s2, s3):
        cid = lax.axis_index("c")
        sid = lax.axis_index("s")
        w = cid * NS + sid
        r0 = sid * ROWS_PER_TILE
        slabs = (i0, i1, i2, i3)
        bufs = (b0, b1, b2, b3)
        gsems = (g0, g1, g2, g3)
        ssems = (s0, s1, s2, s3)
        pltpu.sync_copy(zeros_hbm, agg_sh.at[pl.ds(r0, ROWS_PER_TILE)])
        plsc.subcore_barrier()

        for k in range(GDEPTH):
            pltpu.sync_copy(idx_hbm.at[w, k], slabs[k])
            pltpu.async_copy(xa_hbm.at[slabs[k].at[0]], bufs[k], gsems[k])

        def chunk(c, carry):
            for j in range(NSLOT):
                g = NSLOT * c + j
                t = (j + GDEPTH) % NSLOT

                @pl.when(jnp.logical_and(g >= GDEPTH, g + GDEPTH < NB))
                def _():
                    pltpu.make_async_copy(
                        bufs[t], agg_sh.at[slabs[t].at[1]], ssems[t]).wait()

                @pl.when(g + GDEPTH < NB)
                def _():
                    pltpu.sync_copy(idx_hbm.at[w, g + GDEPTH], slabs[t])
                    pltpu.async_copy(
                        xa_hbm.at[slabs[t].at[0]], bufs[t], gsems[t])

                pltpu.make_async_copy(
                    xa_hbm.at[slabs[j].at[0]], bufs[j], gsems[j]).wait()
                pltpu.async_copy(bufs[j], agg_sh.at[slabs[j].at[1]],
                                 ssems[j], add=True)
            return carry

        lax.fori_loop(0, NB // NSLOT, chunk, 0)
        for k in range(NSLOT):
            pltpu.make_async_copy(
                bufs[k], agg_sh.at[slabs[k].at[1]], ssems[k]).wait()
        plsc.subcore_barrier()
        pltpu.sync_copy(agg_sh.at[pl.ds(r0, ROWS_PER_TILE)],
                        out_hbm.at[cid, pl.ds(r0, ROWS_PER_TILE)])

    return _sc_scatter


_TC_R = 1000


def _tc_body(x_ref, p0_ref, p1_ref, wt_ref, b_ref, o_ref):
    s = p0_ref[0, :, :D_IN] + p1_ref[0, :, :D_IN]
    deg = p0_ref[0, :, D_IN:D_IN + 1] + p1_ref[0, :, D_IN:D_IN + 1]
    agg = s / jnp.maximum(deg, 1.0)
    out = jnp.dot(x_ref[...], wt_ref[:D_IN, :],
                  preferred_element_type=jnp.float32)
    out += jnp.dot(agg, wt_ref[D_IN:, :], preferred_element_type=jnp.float32)
    o_ref[...] = out + b_ref[...]


def kernel(x, edge_index, W, b):
    ei = edge_index.astype(jnp.int32)
    pad = E_TOTAL_PAD - N_EDGES
    dummy_rows = N_NODES + jnp.arange(pad, dtype=jnp.int32) % N_SPARE
    row = jnp.concatenate([ei[0], dummy_rows]).reshape(NW, NB, EDGE_B)
    dummy_cols = jnp.arange(pad, dtype=jnp.int32) * 37 % N_NODES
    col = jnp.concatenate([ei[1], dummy_cols]).reshape(NW, NB, EDGE_B)
    idx = jnp.stack([col, row], axis=2)
    ones_pad = jnp.concatenate(
        [jnp.ones((N_NODES, 1), jnp.float32),
         jnp.zeros((N_NODES, DP - D_IN - 1), jnp.float32)], axis=1)
    xa = jnp.concatenate([x.astype(jnp.float32), ones_pad], axis=1)
    zeros = jnp.zeros((ROWS_PER_TILE, DP), jnp.float32)

    partials = _build_sc_scatter()(xa, idx, zeros)

    wt = W.T.astype(jnp.float32)
    b2 = b.reshape(1, D_OUT).astype(jnp.float32)
    grid = (N_NODES // _TC_R,)
    return pl.pallas_call(
        _tc_body,
        grid=grid,
        in_specs=[
            pl.BlockSpec((_TC_R, D_IN), lambda i: (i, 0)),
            pl.BlockSpec((1, _TC_R, DP), lambda i: (0, i, 0)),
            pl.BlockSpec((1, _TC_R, DP), lambda i: (1, i, 0)),
            pl.BlockSpec((2 * D_IN, D_OUT), lambda i: (0, 0)),
            pl.BlockSpec((1, D_OUT), lambda i: (0, 0)),
        ],
        out_specs=pl.BlockSpec((_TC_R, D_OUT), lambda i: (i, 0)),
        out_shape=jax.ShapeDtypeStruct((N_NODES, D_OUT), jnp.float32),
    )(x.astype(jnp.float32), partials, partials, wt, b2)
